# baseline (speedup 1.0000x reference)
import functools

import jax
import jax.numpy as jnp
from jax import lax
from jax.experimental import pallas as pl
from jax.experimental.pallas import tpu as pltpu
from jax.experimental.pallas import tpu_sc as plsc

_D = 64
_S = 200
_B = 4096
_JB = _B // 128
_UNITS = _S * _JB
_NW = 32
_NE = 1000000
_UPW = _UNITS // _NW


def _body(idxp_hbm, hoff_hbm, table_hbm, out_hbm, idx_all, hoff_all,
          raws, transs, gsems, wsems):
    wid = lax.axis_index("s") * 2 + lax.axis_index("c")
    ubase = wid * _UPW

    pltpu.sync_copy(idxp_hbm.at[pl.ds(ubase * 128, _UPW * 128)], idx_all)
    pltpu.sync_copy(hoff_hbm.at[pl.ds(ubase * 128, _UPW * 128)], hoff_all)

    iota16 = lax.iota(jnp.int32, 16)
    rows16 = [iota16 + (16 * g) for g in range(8)]

    def gather(t, b):
        return pltpu.make_async_copy(
            table_hbm.at[idx_all.at[pl.ds(t * 128, 128)]], raws[b], gsems[b])

    def tile_write(t, b, k):
        v = ubase + t
        s = (v // 256) * 8 + v % 8
        j = (v // 8) % _JB
        return pltpu.make_async_copy(
            transs[b].at[pl.ds(8 * k, 8), :], out_hbm.at[s, k, j], wsems[b])

    gather(0, 0).start()

    def step(o, carry):
        for b in range(2):
            t = o * 2 + b

            @pl.when(t + 1 < _UPW)
            def _():
                gather(t + 1, 1 - b).start()

            gather(t, b).wait()

            @pl.when(t >= 2)
            def _():
                for k in range(8):
                    tile_write(t - 2, b, k).wait()

            raw = raws[b]
            tr = transs[b]

            hvecs = [hoff_all[pl.ds(t * 128 + 16 * g, 16)] for g in range(8)]

            def col(c, carry2):
                for cc in range(4):
                    cvec = (c * 4 + cc + iota16) & (_D - 1)
                    for g in range(8):
                        vec = plsc.load_gather(raw, [rows16[g],
                                                     cvec + hvecs[g]])
                        plsc.store_scatter(tr, [cvec, rows16[g]], vec)
                return carry2

            lax.fori_loop(0, _D // 4, col, 0)

            for k in range(8):
                tile_write(t, b, k).start()
        return carry

    lax.fori_loop(0, _UPW // 2, step, 0)

    for b in range(2):
        for k in range(8):
            tile_write(_UPW - 2 + b, b, k).wait()


def kernel(x, table):
    idx_flat = (x.T.reshape(_S // 8, 8, _JB, 128)
                .transpose(0, 2, 1, 3).reshape(-1))
    idxp = idx_flat >> 1
    hoff = (idx_flat & 1) << 6
    t2 = table.reshape(_NE // 2, 2 * _D)

    mesh = plsc.VectorSubcoreMesh(core_axis_name="c", subcore_axis_name="s")
    run = functools.partial(
        pl.kernel,
        mesh=mesh,
        out_type=jax.ShapeDtypeStruct((_S, 8, _JB, 8, 128), table.dtype),
        scratch_types=[
            pltpu.VMEM((_UPW * 128,), jnp.int32),
            pltpu.VMEM((_UPW * 128,), jnp.int32),
            [pltpu.VMEM((128, 2 * _D), jnp.float32) for _ in range(2)],
            [pltpu.VMEM((_D, 128), jnp.float32) for _ in range(2)],
            [pltpu.SemaphoreType.DMA for _ in range(2)],
            [pltpu.SemaphoreType.DMA for _ in range(2)],
        ],
        compiler_params=pltpu.CompilerParams(
            use_tc_tiling_on_sc=True, needs_layout_passes=False),
    )(_body)
    out5 = run(idxp, hoff, t2)
    return out5.transpose(2, 4, 0, 1, 3).reshape(_B, _S, _D)


# speedup vs baseline: 1.0073x; 1.0073x over previous
import functools

import jax
import jax.numpy as jnp
from jax import lax
from jax.experimental import pallas as pl
from jax.experimental.pallas import tpu as pltpu
from jax.experimental.pallas import tpu_sc as plsc

_D = 64
_S = 200
_B = 4096
_JB = _B // 128
_UNITS = _S * _JB
_NW = 32
_UPW = _UNITS // _NW


def _body(idx_hbm, table_hbm, out_hbm, idx_all, raws, transs, gsems, wsems):
    wid = lax.axis_index("s") * 2 + lax.axis_index("c")
    ubase = wid * _UPW

    pltpu.sync_copy(idx_hbm.at[pl.ds(ubase * 128, _UPW * 128)], idx_all)

    iota16 = lax.iota(jnp.int32, 16)
    rows16 = [iota16 + (16 * g) for g in range(8)]

    def gather(t, b):
        return pltpu.make_async_copy(
            table_hbm.at[idx_all.at[pl.ds(t * 128, 128)]], raws[b], gsems[b])

    def tile_write(t, b, k):
        v = ubase + t
        s = (v // 256) * 8 + v % 8
        j = (v // 8) % _JB
        return pltpu.make_async_copy(
            transs[b].at[pl.ds(8 * k, 8), :], out_hbm.at[s, k, j], wsems[b])

    gather(0, 0).start()

    def step(o, carry):
        for b in range(2):
            t = o * 2 + b

            @pl.when(t + 1 < _UPW)
            def _():
                gather(t + 1, 1 - b).start()

            gather(t, b).wait()

            @pl.when(t >= 2)
            def _():
                for k in range(8):
                    tile_write(t - 2, b, k).wait()

            raw = raws[b]
            tr = transs[b]

            def col(c, carry2):
                for cc in range(4):
                    cvec = (c * 4 + cc + iota16) & (_D - 1)
                    for g in range(8):
                        vec = plsc.load_gather(raw, [rows16[g], cvec])
                        plsc.store_scatter(tr, [cvec, rows16[g]], vec)
                return carry2

            lax.fori_loop(0, _D // 4, col, 0)

            for k in range(8):
                tile_write(t, b, k).start()
        return carry

    lax.fori_loop(0, _UPW // 2, step, 0)

    for b in range(2):
        for k in range(8):
            tile_write(_UPW - 2 + b, b, k).wait()


def kernel(x, table):
    idx_flat = (x.T.reshape(_S // 8, 8, _JB, 128)
                .transpose(0, 2, 1, 3).reshape(-1))

    mesh = plsc.VectorSubcoreMesh(core_axis_name="c", subcore_axis_name="s")
    run = functools.partial(
        pl.kernel,
        mesh=mesh,
        out_type=jax.ShapeDtypeStruct((_S, 8, _JB, 8, 128), table.dtype),
        scratch_types=[
            pltpu.VMEM((_UPW * 128,), jnp.int32),
            [pltpu.VMEM((128, _D), jnp.float32) for _ in range(2)],
            [pltpu.VMEM((_D, 128), jnp.float32) for _ in range(2)],
            [pltpu.SemaphoreType.DMA for _ in range(2)],
            [pltpu.SemaphoreType.DMA for _ in range(2)],
        ],
        compiler_params=pltpu.CompilerParams(
            use_tc_tiling_on_sc=False, needs_layout_passes=False,
            skip_device_barrier=True, disable_bounds_checks=True,
            disable_semaphore_checks=True),
    )(_body)
    out5 = run(idx_flat, table)
    return out5.transpose(2, 4, 0, 1, 3).reshape(_B, _S, _D)
